# baseline (speedup 1.0000x reference)
import functools

import jax
import jax.numpy as jnp
from jax import lax
from jax.experimental import pallas as pl
from jax.experimental.pallas import tpu as pltpu
from jax.experimental.pallas import tpu_sc as plsc

_CHUNK = 256


def _gather_call(table_wide, idx, b_per_w, num_cores):
    B = idx.shape[0]
    D = table_wide.shape[1] // 2
    mesh = plsc.VectorSubcoreMesh(core_axis_name="c", subcore_axis_name="s")

    @functools.partial(
        pl.kernel,
        mesh=mesh,
        out_type=jax.ShapeDtypeStruct((B, D), jnp.float32),
        scratch_types=[
            pltpu.VMEM((b_per_w,), jnp.int32),
            pltpu.VMEM((_CHUNK,), jnp.int32),
            pltpu.VMEM((_CHUNK, 2 * D), jnp.float32),
            pltpu.VMEM((_CHUNK, D), jnp.float32),
            pltpu.SemaphoreType.DMA,
        ],
    )
    def gather_kernel(
        table_hbm, idx_hbm, out_hbm, idx_v, q_v, wide_v, rows_v, sem
    ):
        wid = lax.axis_index("s") * num_cores + lax.axis_index("c")
        base = wid * b_per_w
        pltpu.sync_copy(idx_hbm.at[pl.ds(base, b_per_w)], idx_v)

        for c in range(b_per_w // _CHUNK):
            off = c * _CHUNK

            def pair_ids(g, carry, off=off):
                vec = idx_v[pl.ds(off + g * 16, 16)]
                q_v[pl.ds(g * 16, 16)] = lax.shift_right_logical(vec, 1)
                return carry

            lax.fori_loop(0, _CHUNK // 16, pair_ids, 0)
            pltpu.async_copy(table_hbm.at[q_v], wide_v, sem).wait()

            def select(g, carry, off=off):
                vec = idx_v[pl.ds(off + g * 16, 16)]
                off_vec = lax.mul(lax.bitwise_and(vec, 1), D)
                for lane in range(16):
                    j = g * 16 + lane
                    half = off_vec[lane]
                    for k in range(D // 16):
                        rows_v[j, pl.ds(k * 16, 16)] = wide_v[
                            j, pl.ds(half + k * 16, 16)
                        ]
                return carry

            lax.fori_loop(0, _CHUNK // 16, select, 0)
            pltpu.sync_copy(rows_v, out_hbm.at[pl.ds(base + off, _CHUNK)])

    return gather_kernel(table_wide, idx)


def kernel(expression_face, rand_id):
    info = plsc.get_sparse_core_info()
    nw = info.num_cores * info.num_subcores
    B = rand_id.shape[0]
    b_per_w = B // nw
    table_wide = expression_face.reshape(
        expression_face.shape[0] // 2, 2 * expression_face.shape[1]
    )
    return _gather_call(
        table_wide, rand_id.astype(jnp.int32), b_per_w, info.num_cores
    )
